# baseline (speedup 1.0000x reference)
import functools

import jax
import jax.numpy as jnp
from jax import lax
from jax.experimental import pallas as pl
from jax.experimental.pallas import tpu as pltpu
from jax.experimental.pallas import tpu_sc as plsc

B, S, D = 1024, 200, 128
N = B * S
NP = N // 2
NC, NS = 2, 16
NW = NC * NS
TOK_PER_W = N // NW
CH = 128
CP = CH // 2
NCH = TOK_PER_W // CH
NBUF = 5
NITER = NCH // NBUF
SP = S // 2


def _body(seq_hbm, plab_hbm, tok_hbm, pair_hbm, out_hbm, *rest):
    (seq_v, plab_v, cidx_v, tok_rows, comb_rows, isem, gts, gcs, ws) = (
        rest[k * NBUF:(k + 1) * NBUF] for k in range(9))

    wid = lax.axis_index("s") * NC + lax.axis_index("c")
    w0 = wid * TOK_PER_W
    w0p = wid * (TOK_PER_W // 2)

    def fire_idx(b, c):
        base = w0 + c * CH
        pbase = w0p + c * CP
        pltpu.async_copy(seq_hbm.at[pl.ds(base, CH)], seq_v[b], isem[b])
        pltpu.async_copy(plab_hbm.at[pl.ds(pbase, CP)], plab_v[b], isem[b])

    def fire_gather(b, c):
        base = w0 + c * CH
        pltpu.make_async_copy(seq_hbm.at[pl.ds(0, CH)], seq_v[b], isem[b]).wait()
        pltpu.make_async_copy(plab_hbm.at[pl.ds(0, CP)], plab_v[b], isem[b]).wait()
        for q in range(CP // 16):
            sl = pl.ds(q * 16, 16)
            u = lax.rem(w0p + c * CP + q * 16 + lax.iota(jnp.int32, 16), SP)
            cidx_v[b][sl] = plab_v[b][sl] * SP + u + lax.rem(wid, 8) * (9 * SP)
        pltpu.async_copy(tok_hbm.at[seq_v[b]], tok_rows[b], gts[b])
        pltpu.async_copy(pair_hbm.at[cidx_v[b]], comb_rows[b], gcs[b])

    def consume(b, c):
        base = w0 + c * CH
        pltpu.make_async_copy(tok_hbm.at[seq_v[b]], tok_rows[b], gts[b]).wait()
        pltpu.make_async_copy(pair_hbm.at[cidx_v[b]], comb_rows[b], gcs[b]).wait()

        def pair_body(t, acc):
            for m in range(2):
                r = 2 * t + m
                for g in range(D // 32):
                    w = comb_rows[b][t, pl.ds(m * 64 + g * 16, 16)]
                    lo = lax.bitcast_convert_type(
                        lax.shift_left(w, 16), jnp.float32)
                    hi = lax.bitcast_convert_type(
                        w & jnp.int32(-65536), jnp.float32)
                    plsc.addupdate(tok_rows[b].at[r, pl.ds(g * 32, 16)], lo)
                    plsc.addupdate(
                        tok_rows[b].at[r, pl.ds(g * 32 + 16, 16)], hi)
            return acc

        lax.fori_loop(0, CP, pair_body, 0)
        pltpu.async_copy(tok_rows[b], out_hbm.at[pl.ds(base, CH)], ws[b])

    def wait_wb(b):
        pltpu.make_async_copy(tok_rows[b], out_hbm.at[pl.ds(0, CH)], ws[b]).wait()

    for c in range(3):
        fire_idx(c % NBUF, c)
    for c in range(2):
        fire_gather(c % NBUF, c)

    def step(i, carry):
        for j in range(NBUF):
            c = i * NBUF + j
            bf2 = (j + 2) % NBUF
            bf3 = (j + 3) % NBUF
            if j >= NBUF - 2:
                wait_wb(bf2)
            else:
                pl.when(i >= 1)(lambda bb=bf2: wait_wb(bb))
            if j <= NBUF - 4:
                fire_idx(bf3, c + 3)
            else:
                pl.when(i < NITER - 1)(lambda bb=bf3, cc=c + 3: fire_idx(bb, cc))
            if j <= NBUF - 3:
                fire_gather(bf2, c + 2)
            else:
                pl.when(i < NITER - 1)(lambda bb=bf2, cc=c + 2: fire_gather(bb, cc))
            consume(j, c)
        return carry

    lax.fori_loop(0, NITER, step, 0)
    for k in range(NCH - (NBUF - 2), NCH):
        wait_wb(k % NBUF)


@functools.partial(
    pl.kernel,
    out_type=jax.ShapeDtypeStruct((N, D), jnp.float32),
    mesh=plsc.VectorSubcoreMesh(core_axis_name="c", subcore_axis_name="s"),
    scratch_types=(
        [pltpu.VMEM((CH,), jnp.int32) for _ in range(NBUF)]
        + [pltpu.VMEM((CP,), jnp.int32) for _ in range(NBUF)]
        + [pltpu.VMEM((CP,), jnp.int32) for _ in range(NBUF)]
        + [pltpu.VMEM((CH, D), jnp.float32) for _ in range(NBUF)]
        + [pltpu.VMEM((CP, D), jnp.int32) for _ in range(NBUF)]
        + [pltpu.SemaphoreType.DMA for _ in range(4 * NBUF)]
    ),
)
def _sc_embed(*args):
    _body(*args)


def kernel(sequence, segment_label, token_table, segment_table, pe):
    comb = (segment_table[:, None, :] + pe[0, :S, :][None, :, :]).reshape(3 * S, D)
    packed = comb.reshape(3 * S, D // 32, 2, 16).transpose(0, 1, 3, 2)
    packed = packed.reshape(3 * S, D // 2, 2).astype(jnp.bfloat16)
    packed = lax.bitcast_convert_type(packed, jnp.int32)
    pk = packed.reshape(3, S, D // 2)
    even, odd = pk[:, 0::2, :], pk[:, 1::2, :]
    pair = jnp.concatenate(
        [jnp.broadcast_to(even[:, None], (3, 3, SP, D // 2)),
         jnp.broadcast_to(odd[None, :], (3, 3, SP, D // 2))],
        axis=-1).reshape(9 * SP, D)
    pair = jnp.tile(pair, (8, 1))
    lab2 = segment_label.reshape(B, SP, 2)
    plab = (lab2[..., 0] * 3 + lab2[..., 1]).reshape(NP).astype(jnp.int32)
    out = _sc_embed(sequence.reshape(N), plab, token_table, pair)
    return out.reshape(B, S, D)


# speedup vs baseline: 1.0077x; 1.0077x over previous
import functools

import jax
import jax.numpy as jnp
from jax import lax
from jax.experimental import pallas as pl
from jax.experimental.pallas import tpu as pltpu
from jax.experimental.pallas import tpu_sc as plsc

B, S, D = 1024, 200, 128
N = B * S
NP = N // 2
NC, NS = 2, 16
NW = NC * NS
TOK_PER_W = N // NW
CH = 128
CP = CH // 2
NCH = TOK_PER_W // CH
NBUF = 5
NITER = NCH // NBUF
SP = S // 2


def _body(seq_hbm, plab_hbm, tok_hbm, pair_hbm, out_hbm, *rest):
    (seq_v, plab_v, cidx_v, tok_rows, comb_rows, isem, gts, gcs, ws) = (
        rest[k * NBUF:(k + 1) * NBUF] for k in range(9))

    wid = lax.axis_index("s") * NC + lax.axis_index("c")
    w0 = wid * TOK_PER_W
    w0p = wid * (TOK_PER_W // 2)

    def fire_idx(b, c):
        base = w0 + c * CH
        pbase = w0p + c * CP
        pltpu.async_copy(seq_hbm.at[pl.ds(base, CH)], seq_v[b], isem[b])
        pltpu.async_copy(plab_hbm.at[pl.ds(pbase, CP)], plab_v[b], isem[b])

    def fire_gather(b, c):
        base = w0 + c * CH
        pltpu.make_async_copy(seq_hbm.at[pl.ds(0, CH)], seq_v[b], isem[b]).wait()
        pltpu.make_async_copy(plab_hbm.at[pl.ds(0, CP)], plab_v[b], isem[b]).wait()
        for q in range(CP // 16):
            sl = pl.ds(q * 16, 16)
            u = lax.rem(w0p + c * CP + q * 16 + lax.iota(jnp.int32, 16), SP)
            cidx_v[b][sl] = plab_v[b][sl] * SP + u
        pltpu.async_copy(tok_hbm.at[seq_v[b]], tok_rows[b], gts[b])
        pltpu.async_copy(pair_hbm.at[cidx_v[b]], comb_rows[b], gcs[b])

    def consume(b, c):
        base = w0 + c * CH
        pltpu.make_async_copy(tok_hbm.at[seq_v[b]], tok_rows[b], gts[b]).wait()
        pltpu.make_async_copy(pair_hbm.at[cidx_v[b]], comb_rows[b], gcs[b]).wait()

        def pair_body(t, acc):
            for m in range(2):
                r = 2 * t + m
                for g in range(D // 32):
                    w = comb_rows[b][t, pl.ds(m * 64 + g * 16, 16)]
                    lo = lax.bitcast_convert_type(
                        lax.shift_left(w, 16), jnp.float32)
                    hi = lax.bitcast_convert_type(
                        w & jnp.int32(-65536), jnp.float32)
                    plsc.addupdate(tok_rows[b].at[r, pl.ds(g * 32, 16)], lo)
                    plsc.addupdate(
                        tok_rows[b].at[r, pl.ds(g * 32 + 16, 16)], hi)
            return acc

        lax.fori_loop(0, CP, pair_body, 0)
        pltpu.async_copy(tok_rows[b], out_hbm.at[pl.ds(base, CH)], ws[b])

    def wait_wb(b):
        pltpu.make_async_copy(tok_rows[b], out_hbm.at[pl.ds(0, CH)], ws[b]).wait()

    for c in range(3):
        fire_idx(c % NBUF, c)
    for c in range(2):
        fire_gather(c % NBUF, c)

    def step(i, carry):
        for j in range(NBUF):
            c = i * NBUF + j
            bf2 = (j + 2) % NBUF
            bf3 = (j + 3) % NBUF
            if j >= NBUF - 2:
                wait_wb(bf2)
            else:
                pl.when(i >= 1)(lambda bb=bf2: wait_wb(bb))
            if j <= NBUF - 4:
                fire_idx(bf3, c + 3)
            else:
                pl.when(i < NITER - 1)(lambda bb=bf3, cc=c + 3: fire_idx(bb, cc))
            if j <= NBUF - 3:
                fire_gather(bf2, c + 2)
            else:
                pl.when(i < NITER - 1)(lambda bb=bf2, cc=c + 2: fire_gather(bb, cc))
            consume(j, c)
        return carry

    lax.fori_loop(0, NITER, step, 0)
    for k in range(NCH - (NBUF - 2), NCH):
        wait_wb(k % NBUF)


@functools.partial(
    pl.kernel,
    out_type=jax.ShapeDtypeStruct((N, D), jnp.float32),
    mesh=plsc.VectorSubcoreMesh(core_axis_name="c", subcore_axis_name="s"),
    scratch_types=(
        [pltpu.VMEM((CH,), jnp.int32) for _ in range(NBUF)]
        + [pltpu.VMEM((CP,), jnp.int32) for _ in range(NBUF)]
        + [pltpu.VMEM((CP,), jnp.int32) for _ in range(NBUF)]
        + [pltpu.VMEM((CH, D), jnp.float32) for _ in range(NBUF)]
        + [pltpu.VMEM((CP, D), jnp.int32) for _ in range(NBUF)]
        + [pltpu.SemaphoreType.DMA for _ in range(4 * NBUF)]
    ),
)
def _sc_embed(*args):
    _body(*args)


def kernel(sequence, segment_label, token_table, segment_table, pe):
    comb = (segment_table[:, None, :] + pe[0, :S, :][None, :, :]).reshape(3 * S, D)
    packed = comb.reshape(3 * S, D // 32, 2, 16).transpose(0, 1, 3, 2)
    packed = packed.reshape(3 * S, D // 2, 2).astype(jnp.bfloat16)
    packed = lax.bitcast_convert_type(packed, jnp.int32)
    pk = packed.reshape(3, S, D // 2)
    even, odd = pk[:, 0::2, :], pk[:, 1::2, :]
    pair = jnp.concatenate(
        [jnp.broadcast_to(even[:, None], (3, 3, SP, D // 2)),
         jnp.broadcast_to(odd[None, :], (3, 3, SP, D // 2))],
        axis=-1).reshape(9 * SP, D)
    lab2 = segment_label.reshape(B, SP, 2)
    plab = (lab2[..., 0] * 3 + lab2[..., 1]).reshape(NP).astype(jnp.int32)
    out = _sc_embed(sequence.reshape(N), plab, token_table, pair)
    return out.reshape(B, S, D)
